# baseline (speedup 1.0000x reference)
import functools

import jax
import jax.numpy as jnp
from jax.experimental import pallas as pl
from jax.experimental.pallas import tpu as pltpu

_DSM = 0.01
_BIG = 3.0e38
_LANES = 128


def _top4_values(d2, n_slices):
    R = d2.shape[0]
    m1 = jnp.full((R, _LANES), _BIG, jnp.float32)
    m2, m3, m4 = m1, m1, m1
    for s in range(n_slices):
        x = d2[:, s * _LANES:(s + 1) * _LANES]
        o1 = jnp.maximum(m1, x)
        m1 = jnp.minimum(m1, x)
        o2 = jnp.maximum(m2, o1)
        m2 = jnp.minimum(m2, o1)
        o3 = jnp.maximum(m3, o2)
        m3 = jnp.minimum(m3, o2)
        m4 = jnp.minimum(m4, o3)
    cat = jnp.concatenate([m1, m2, m3, m4], axis=1)
    iota_cat = jax.lax.broadcasted_iota(jnp.int32, cat.shape, 1)
    vs = []
    for j in range(4):
        mj = jnp.min(cat, axis=1, keepdims=True)
        vs.append(mj)
        if j < 3:
            isel = jnp.min(jnp.where(cat <= mj, iota_cat, jnp.int32(2**30)),
                           axis=1, keepdims=True)
            cat = jnp.where(iota_cat == isel, _BIG, cat)
    return vs


def _lanscore_kernel(
    pts_n_ref, ptsT_n_ref, pts_c_ref, ptsT_c_ref, pnt_oh_ref,
    W1_ref, b1_ref, W2_ref, b2_ref, Wx_ref, Wc_ref, b0_ref,
    Wblk_ref, bblk_ref, Wout_ref, bout_ref,
    out_ref,
    frames_scr, noise_scr,
    *, n_valid, P, K, C, Q_CHUNK,
):
    pts_n = pts_n_ref[0]
    ptsT_n = ptsT_n_ref[0]
    pts_c = pts_c_ref[0]
    ptsT_c = ptsT_c_ref[0]
    npad = pts_n.shape[0]
    m_clean = pts_c.shape[0]

    pnt_oh = pnt_oh_ref[...]
    q = jnp.dot(pnt_oh, pts_n, preferred_element_type=jnp.float32)
    h1 = jnp.maximum(
        jnp.dot(q, W1_ref[...], preferred_element_type=jnp.float32)
        + b1_ref[...], 0.0)
    feat = (jnp.dot(h1, W2_ref[...], preferred_element_type=jnp.float32)
            + b2_ref[...])

    pp_n = jnp.sum(ptsT_n * ptsT_n, axis=0, keepdims=True)
    col1_n = jax.lax.broadcasted_iota(jnp.int32, (1, npad), 1)
    if n_valid < npad:
        pp_n = jnp.where(col1_n >= n_valid, _BIG, pp_n)
    ptsT_na = jnp.concatenate([ptsT_n, pp_n], axis=0)
    qa = jnp.concatenate([q * -2.0, jnp.ones((P, 1), jnp.float32)], axis=1)
    d2n = jnp.dot(qa, ptsT_na, preferred_element_type=jnp.float32)

    n_slices_n = npad // _LANES
    for rnd in range(K // 4):
        vs = _top4_values(d2n, n_slices_n)
        for j in range(4):
            ohf = (d2n == vs[j]).astype(jnp.float32)
            frames_scr[(rnd * 4 + j) * P:(rnd * 4 + j + 1) * P, :] = jnp.dot(
                ohf, pts_n, preferred_element_type=jnp.float32)
        d2n = jnp.where(d2n <= vs[3], _BIG, d2n)
    frames = frames_scr[...]

    pp_c = jnp.sum(ptsT_c * ptsT_c, axis=0, keepdims=True)
    ptsT_ca = jnp.concatenate([ptsT_c, pp_c], axis=0)
    n_rows = K * P
    n_slices = m_clean // _LANES

    n_chunks = n_rows // Q_CHUNK

    def _dist_chunk(ci):
        fc = frames_scr[pl.ds(ci * Q_CHUNK, Q_CHUNK), :]
        fca = jnp.concatenate(
            [fc * -2.0, jnp.ones((Q_CHUNK, 1), jnp.float32)], axis=1)
        return fc, jnp.dot(fca, ptsT_ca,
                           preferred_element_type=jnp.float32)

    def _chunk_body(ci, carry):
        fc, d2c = carry
        next_carry = _dist_chunk(jnp.minimum(ci + 1, n_chunks - 1))
        v4 = _top4_values(d2c, n_slices)[3]
        ohf = (d2c <= v4).astype(jnp.float32)
        cnt = jnp.sum(ohf, axis=1, keepdims=True)
        acc = jnp.dot(ohf, pts_c, preferred_element_type=jnp.float32)
        noise_scr[pl.ds(ci * Q_CHUNK, Q_CHUNK), :] = fc - acc / cnt
        return next_carry

    jax.lax.fori_loop(0, n_chunks, _chunk_body, _dist_chunk(0))
    noise = noise_scr[...]

    q_t = jnp.concatenate([q] * K, axis=0)
    x = frames - q_t
    cwc = jnp.dot(feat, Wc_ref[...], preferred_element_type=jnp.float32)
    cwc_t = jnp.concatenate([cwc] * K, axis=0)
    h = jnp.maximum(
        jnp.dot(x, Wx_ref[...], preferred_element_type=jnp.float32)
        + cwc_t + b0_ref[...], 0.0)
    for i in range(Wblk_ref.shape[0]):
        h = jnp.maximum(
            jnp.dot(h, Wblk_ref[i], preferred_element_type=jnp.float32)
            + bblk_ref[i][None, :], 0.0) + h
    gp = (jnp.dot(h, Wout_ref[...], preferred_element_type=jnp.float32)
          + bout_ref[...])

    diff = -noise - gp
    s = jnp.sum(diff * diff).reshape(1, 1)
    @pl.when(pl.program_id(0) == 0)
    def _():
        out_ref[...] = jnp.zeros_like(out_ref)
    out_ref[...] += s


def kernel(pcl_noisy, pcl_clean, W1, b1, W2, b2, Wx, Wc, b0, Wblk, bblk,
           Wout, bout, pnt_idx):
    B, N, _ = pcl_noisy.shape
    M = pcl_clean.shape[1]
    P = pnt_idx.shape[0]
    K = 32
    C = 4
    Q_CHUNK = 64

    npad = ((N + _LANES - 1) // _LANES) * _LANES
    pts_n = jnp.pad(pcl_noisy, ((0, 0), (0, npad - N), (0, 0)))
    ptsT_n = jnp.transpose(pts_n, (0, 2, 1))
    pts_c = pcl_clean
    ptsT_c = jnp.transpose(pcl_clean, (0, 2, 1))
    pnt_oh = (pnt_idx[:, None] ==
              jnp.arange(npad, dtype=pnt_idx.dtype)[None, :]
              ).astype(jnp.float32)

    bspec = lambda shp, imap: pl.BlockSpec(shp, imap)
    per_b = lambda *shp: pl.BlockSpec((1,) + shp, lambda b: (b,) + (0,) * len(shp))
    fixed = lambda *shp: pl.BlockSpec(shp, lambda b: (0,) * len(shp))

    out = pl.pallas_call(
        functools.partial(_lanscore_kernel, n_valid=N, P=P, K=K, C=C,
                          Q_CHUNK=Q_CHUNK),
        grid=(B,),
        in_specs=[
            per_b(npad, 3), per_b(3, npad), per_b(M, 3), per_b(3, M),
            fixed(P, npad),
            fixed(3, W1.shape[1]), fixed(1, b1.shape[0]),
            fixed(*W2.shape), fixed(1, b2.shape[0]),
            fixed(*Wx.shape), fixed(*Wc.shape), fixed(1, b0.shape[0]),
            fixed(*Wblk.shape), fixed(*bblk.shape),
            fixed(*Wout.shape), fixed(1, bout.shape[0]),
        ],
        out_specs=pl.BlockSpec((1, 1), lambda b: (0, 0)),
        out_shape=jax.ShapeDtypeStruct((1, 1), jnp.float32),
        scratch_shapes=[
            pltpu.VMEM((K * P, 3), jnp.float32),
            pltpu.VMEM((K * P, 3), jnp.float32),
        ],
        compiler_params=pltpu.CompilerParams(
            dimension_semantics=("arbitrary",),
        ),
    )(
        pts_n, ptsT_n, pts_c, ptsT_c, pnt_oh,
        W1, b1[None, :], W2, b2[None, :], Wx, Wc, b0[None, :],
        Wblk, bblk, Wout, bout[None, :],
    )
    scale = 0.5 / (_DSM * B * P * K)
    return (out[0, 0] * scale).astype(jnp.float32)


# speedup vs baseline: 1.1391x; 1.1391x over previous
import functools

import jax
import jax.numpy as jnp
from jax.experimental import pallas as pl
from jax.experimental.pallas import tpu as pltpu

_DSM = 0.01
_BIG = 3.0e38
_LANES = 128


def _top4_values(d2, n_slices):
    R = d2.shape[0]
    m1 = jnp.full((R, _LANES), _BIG, jnp.float32)
    m2, m3, m4 = m1, m1, m1
    for s in range(n_slices):
        x = d2[:, s * _LANES:(s + 1) * _LANES]
        o1 = jnp.maximum(m1, x)
        m1 = jnp.minimum(m1, x)
        o2 = jnp.maximum(m2, o1)
        m2 = jnp.minimum(m2, o1)
        o3 = jnp.maximum(m3, o2)
        m3 = jnp.minimum(m3, o2)
        m4 = jnp.minimum(m4, o3)
    cat = jnp.concatenate([m1, m2, m3, m4], axis=1)
    iota_cat = jax.lax.broadcasted_iota(jnp.int32, cat.shape, 1)
    vs = []
    for j in range(4):
        mj = jnp.min(cat, axis=1, keepdims=True)
        vs.append(mj)
        if j < 3:
            isel = jnp.min(jnp.where(cat <= mj, iota_cat, jnp.int32(2**30)),
                           axis=1, keepdims=True)
            cat = jnp.where(iota_cat == isel, _BIG, cat)
    return vs


def _lanscore_kernel(
    pts_n_ref, ptsT_n_ref, pts_c_ref, ptsT_c_ref, pnt_oh_ref,
    W1_ref, b1_ref, W2_ref, b2_ref, Wx_ref, Wc_ref, b0_ref,
    Wblk_ref, bblk_ref, Wout_ref, bout_ref,
    out_ref,
    frames_scr, noise_scr,
    *, n_valid, P, K, C, Q_CHUNK,
):
    pts_n = pts_n_ref[0]
    ptsT_n = ptsT_n_ref[0]
    pts_c = pts_c_ref[0]
    ptsT_c = ptsT_c_ref[0]
    npad = pts_n.shape[0]
    m_clean = pts_c.shape[0]

    pnt_oh = pnt_oh_ref[...]
    q = jnp.dot(pnt_oh, pts_n, preferred_element_type=jnp.float32)
    h1 = jnp.maximum(
        jnp.dot(q, W1_ref[...], preferred_element_type=jnp.float32)
        + b1_ref[...], 0.0)
    feat = (jnp.dot(h1, W2_ref[...], preferred_element_type=jnp.float32)
            + b2_ref[...])

    pp_n = jnp.sum(ptsT_n * ptsT_n, axis=0, keepdims=True)
    col1_n = jax.lax.broadcasted_iota(jnp.int32, (1, npad), 1)
    if n_valid < npad:
        pp_n = jnp.where(col1_n >= n_valid, _BIG, pp_n)
    ptsT_na = jnp.concatenate([ptsT_n, pp_n], axis=0)
    qa = jnp.concatenate([q * -2.0, jnp.ones((P, 1), jnp.float32)], axis=1)
    d2n = jnp.dot(qa, ptsT_na, preferred_element_type=jnp.float32)

    n_slices_n = npad // _LANES
    for rnd in range(K // 4):
        vs = _top4_values(d2n, n_slices_n)
        ohf4 = jnp.concatenate(
            [(d2n == v).astype(jnp.float32) for v in vs], axis=0)
        frames_scr[rnd * 4 * P:(rnd + 1) * 4 * P, :] = jnp.dot(
            ohf4, pts_n, preferred_element_type=jnp.float32)
        d2n = jnp.where(d2n <= vs[3], _BIG, d2n)
    frames = frames_scr[...]

    pp_c = jnp.sum(ptsT_c * ptsT_c, axis=0, keepdims=True)
    ptsT_ca = jnp.concatenate([ptsT_c, pp_c], axis=0)
    n_rows = K * P
    n_slices = m_clean // _LANES

    def _chunk_body(ci, carry):
        fc = frames_scr[pl.ds(ci * Q_CHUNK, Q_CHUNK), :]
        fca = jnp.concatenate(
            [fc * -2.0, jnp.ones((Q_CHUNK, 1), jnp.float32)], axis=1)
        d2c = jnp.dot(fca, ptsT_ca,
                      preferred_element_type=jnp.float32)
        v4 = _top4_values(d2c, n_slices)[3]
        ohf = (d2c <= v4).astype(jnp.float32)
        cnt = jnp.sum(ohf, axis=1, keepdims=True)
        acc = jnp.dot(ohf, pts_c, preferred_element_type=jnp.float32)
        noise_scr[pl.ds(ci * Q_CHUNK, Q_CHUNK), :] = fc - acc / cnt
        return carry

    jax.lax.fori_loop(0, n_rows // Q_CHUNK, _chunk_body, 0)
    noise = noise_scr[...]

    q_t = jnp.concatenate([q] * K, axis=0)
    x = frames - q_t
    cwc = jnp.dot(feat, Wc_ref[...], preferred_element_type=jnp.float32)
    cwc_t = jnp.concatenate([cwc] * K, axis=0)
    h = jnp.maximum(
        jnp.dot(x, Wx_ref[...], preferred_element_type=jnp.float32)
        + cwc_t + b0_ref[...], 0.0)
    for i in range(Wblk_ref.shape[0]):
        h = jnp.maximum(
            jnp.dot(h, Wblk_ref[i], preferred_element_type=jnp.float32)
            + bblk_ref[i][None, :], 0.0) + h
    gp = (jnp.dot(h, Wout_ref[...], preferred_element_type=jnp.float32)
          + bout_ref[...])

    diff = -noise - gp
    s = jnp.sum(diff * diff).reshape(1, 1)
    @pl.when(pl.program_id(0) == 0)
    def _():
        out_ref[...] = jnp.zeros_like(out_ref)
    out_ref[...] += s


def kernel(pcl_noisy, pcl_clean, W1, b1, W2, b2, Wx, Wc, b0, Wblk, bblk,
           Wout, bout, pnt_idx):
    B, N, _ = pcl_noisy.shape
    M = pcl_clean.shape[1]
    P = pnt_idx.shape[0]
    K = 32
    C = 4
    Q_CHUNK = 64

    npad = ((N + _LANES - 1) // _LANES) * _LANES
    pts_n = jnp.pad(pcl_noisy, ((0, 0), (0, npad - N), (0, 0)))
    ptsT_n = jnp.transpose(pts_n, (0, 2, 1))
    pts_c = pcl_clean
    ptsT_c = jnp.transpose(pcl_clean, (0, 2, 1))
    pnt_oh = (pnt_idx[:, None] ==
              jnp.arange(npad, dtype=pnt_idx.dtype)[None, :]
              ).astype(jnp.float32)

    bspec = lambda shp, imap: pl.BlockSpec(shp, imap)
    per_b = lambda *shp: pl.BlockSpec((1,) + shp, lambda b: (b,) + (0,) * len(shp))
    fixed = lambda *shp: pl.BlockSpec(shp, lambda b: (0,) * len(shp))

    out = pl.pallas_call(
        functools.partial(_lanscore_kernel, n_valid=N, P=P, K=K, C=C,
                          Q_CHUNK=Q_CHUNK),
        grid=(B,),
        in_specs=[
            per_b(npad, 3), per_b(3, npad), per_b(M, 3), per_b(3, M),
            fixed(P, npad),
            fixed(3, W1.shape[1]), fixed(1, b1.shape[0]),
            fixed(*W2.shape), fixed(1, b2.shape[0]),
            fixed(*Wx.shape), fixed(*Wc.shape), fixed(1, b0.shape[0]),
            fixed(*Wblk.shape), fixed(*bblk.shape),
            fixed(*Wout.shape), fixed(1, bout.shape[0]),
        ],
        out_specs=pl.BlockSpec((1, 1), lambda b: (0, 0)),
        out_shape=jax.ShapeDtypeStruct((1, 1), jnp.float32),
        scratch_shapes=[
            pltpu.VMEM((K * P, 3), jnp.float32),
            pltpu.VMEM((K * P, 3), jnp.float32),
        ],
        compiler_params=pltpu.CompilerParams(
            dimension_semantics=("arbitrary",),
        ),
    )(
        pts_n, ptsT_n, pts_c, ptsT_c, pnt_oh,
        W1, b1[None, :], W2, b2[None, :], Wx, Wc, b0[None, :],
        Wblk, bblk, Wout, bout[None, :],
    )
    scale = 0.5 / (_DSM * B * P * K)
    return (out[0, 0] * scale).astype(jnp.float32)


# speedup vs baseline: 1.1412x; 1.0018x over previous
import functools

import jax
import jax.numpy as jnp
from jax.experimental import pallas as pl
from jax.experimental.pallas import tpu as pltpu

_DSM = 0.01
_BIG = 3.0e38
_LANES = 128


def _top4_values(d2, n_slices):
    R = d2.shape[0]
    m1 = jnp.full((R, _LANES), _BIG, jnp.float32)
    m2, m3, m4 = m1, m1, m1
    for s in range(n_slices):
        x = d2[:, s * _LANES:(s + 1) * _LANES]
        o1 = jnp.maximum(m1, x)
        m1 = jnp.minimum(m1, x)
        o2 = jnp.maximum(m2, o1)
        m2 = jnp.minimum(m2, o1)
        o3 = jnp.maximum(m3, o2)
        m3 = jnp.minimum(m3, o2)
        m4 = jnp.minimum(m4, o3)
    cat = jnp.concatenate([m1, m2, m3, m4], axis=1)
    iota_cat = jax.lax.broadcasted_iota(jnp.int32, cat.shape, 1)
    vs = []
    for j in range(4):
        mj = jnp.min(cat, axis=1, keepdims=True)
        vs.append(mj)
        if j < 3:
            isel = jnp.min(jnp.where(cat <= mj, iota_cat, jnp.int32(2**30)),
                           axis=1, keepdims=True)
            cat = jnp.where(iota_cat == isel, _BIG, cat)
    return vs


def _lanscore_kernel(
    pts_n_ref, ptsT_n_ref, pts_c_ref, ptsT_c_ref, pnt_oh_ref,
    W1_ref, b1_ref, W2_ref, b2_ref, Wx_ref, Wc_ref, b0_ref,
    Wblk_ref, bblk_ref, Wout_ref, bout_ref,
    out_ref,
    frames_scr, noise_scr,
    *, n_valid, P, K, C, Q_CHUNK,
):
    pts_n = pts_n_ref[0]
    ptsT_n = ptsT_n_ref[0]
    pts_c = pts_c_ref[0]
    ptsT_c = ptsT_c_ref[0]
    npad = pts_n.shape[0]
    m_clean = pts_c.shape[0]

    pnt_oh = pnt_oh_ref[...]
    q = jnp.dot(pnt_oh, pts_n, preferred_element_type=jnp.float32)
    h1 = jnp.maximum(
        jnp.dot(q, W1_ref[...], preferred_element_type=jnp.float32)
        + b1_ref[...], 0.0)
    feat = (jnp.dot(h1, W2_ref[...], preferred_element_type=jnp.float32)
            + b2_ref[...])

    pp_n = jnp.sum(ptsT_n * ptsT_n, axis=0, keepdims=True)
    col1_n = jax.lax.broadcasted_iota(jnp.int32, (1, npad), 1)
    if n_valid < npad:
        pp_n = jnp.where(col1_n >= n_valid, _BIG, pp_n)
    ptsT_na = jnp.concatenate([ptsT_n, pp_n], axis=0)
    qa = jnp.concatenate([q * -2.0, jnp.ones((P, 1), jnp.float32)], axis=1)
    d2n = jnp.dot(qa, ptsT_na, preferred_element_type=jnp.float32)

    n_slices_n = npad // _LANES
    for rnd in range(K // 4):
        vs = _top4_values(d2n, n_slices_n)
        ohf4 = jnp.concatenate(
            [(d2n == v).astype(jnp.float32) for v in vs], axis=0)
        frames_scr[rnd * 4 * P:(rnd + 1) * 4 * P, :] = jnp.dot(
            ohf4, pts_n, preferred_element_type=jnp.float32)
        d2n = jnp.where(d2n <= vs[3], _BIG, d2n)
    frames = frames_scr[...]

    pp_c = jnp.sum(ptsT_c * ptsT_c, axis=0, keepdims=True)
    ptsT_ca = jnp.concatenate([ptsT_c, pp_c], axis=0)
    pts_c1 = jnp.concatenate(
        [pts_c, jnp.ones((m_clean, 1), jnp.float32)], axis=1)
    n_rows = K * P
    n_slices = m_clean // _LANES

    def _chunk_body(ci, carry):
        fc = frames_scr[pl.ds(ci * Q_CHUNK, Q_CHUNK), :]
        fca = jnp.concatenate(
            [fc * -2.0, jnp.ones((Q_CHUNK, 1), jnp.float32)], axis=1)
        d2c = jnp.dot(fca, ptsT_ca,
                      preferred_element_type=jnp.float32)
        v4 = _top4_values(d2c, n_slices)[3]
        ohf = (d2c <= v4).astype(jnp.float32)
        acc = jnp.dot(ohf, pts_c1, preferred_element_type=jnp.float32)
        noise_scr[pl.ds(ci * Q_CHUNK, Q_CHUNK), :] = (
            fc - acc[:, :3] / acc[:, 3:4])
        return carry

    jax.lax.fori_loop(0, n_rows // Q_CHUNK, _chunk_body, 0)
    noise = noise_scr[...]

    q_t = jnp.concatenate([q] * K, axis=0)
    x = frames - q_t
    cwc = jnp.dot(feat, Wc_ref[...], preferred_element_type=jnp.float32)
    cwc_t = jnp.concatenate([cwc] * K, axis=0)
    h = jnp.maximum(
        jnp.dot(x, Wx_ref[...], preferred_element_type=jnp.float32)
        + cwc_t + b0_ref[...], 0.0)
    for i in range(Wblk_ref.shape[0]):
        h = jnp.maximum(
            jnp.dot(h, Wblk_ref[i], preferred_element_type=jnp.float32)
            + bblk_ref[i][None, :], 0.0) + h
    gp = (jnp.dot(h, Wout_ref[...], preferred_element_type=jnp.float32)
          + bout_ref[...])

    diff = -noise - gp
    s = jnp.sum(diff * diff).reshape(1, 1)
    @pl.when(pl.program_id(0) == 0)
    def _():
        out_ref[...] = jnp.zeros_like(out_ref)
    out_ref[...] += s


def kernel(pcl_noisy, pcl_clean, W1, b1, W2, b2, Wx, Wc, b0, Wblk, bblk,
           Wout, bout, pnt_idx):
    B, N, _ = pcl_noisy.shape
    M = pcl_clean.shape[1]
    P = pnt_idx.shape[0]
    K = 32
    C = 4
    Q_CHUNK = 64

    npad = ((N + _LANES - 1) // _LANES) * _LANES
    pts_n = jnp.pad(pcl_noisy, ((0, 0), (0, npad - N), (0, 0)))
    ptsT_n = jnp.transpose(pts_n, (0, 2, 1))
    pts_c = pcl_clean
    ptsT_c = jnp.transpose(pcl_clean, (0, 2, 1))
    pnt_oh = (pnt_idx[:, None] ==
              jnp.arange(npad, dtype=pnt_idx.dtype)[None, :]
              ).astype(jnp.float32)

    bspec = lambda shp, imap: pl.BlockSpec(shp, imap)
    per_b = lambda *shp: pl.BlockSpec((1,) + shp, lambda b: (b,) + (0,) * len(shp))
    fixed = lambda *shp: pl.BlockSpec(shp, lambda b: (0,) * len(shp))

    out = pl.pallas_call(
        functools.partial(_lanscore_kernel, n_valid=N, P=P, K=K, C=C,
                          Q_CHUNK=Q_CHUNK),
        grid=(B,),
        in_specs=[
            per_b(npad, 3), per_b(3, npad), per_b(M, 3), per_b(3, M),
            fixed(P, npad),
            fixed(3, W1.shape[1]), fixed(1, b1.shape[0]),
            fixed(*W2.shape), fixed(1, b2.shape[0]),
            fixed(*Wx.shape), fixed(*Wc.shape), fixed(1, b0.shape[0]),
            fixed(*Wblk.shape), fixed(*bblk.shape),
            fixed(*Wout.shape), fixed(1, bout.shape[0]),
        ],
        out_specs=pl.BlockSpec((1, 1), lambda b: (0, 0)),
        out_shape=jax.ShapeDtypeStruct((1, 1), jnp.float32),
        scratch_shapes=[
            pltpu.VMEM((K * P, 3), jnp.float32),
            pltpu.VMEM((K * P, 3), jnp.float32),
        ],
        compiler_params=pltpu.CompilerParams(
            dimension_semantics=("arbitrary",),
        ),
    )(
        pts_n, ptsT_n, pts_c, ptsT_c, pnt_oh,
        W1, b1[None, :], W2, b2[None, :], Wx, Wc, b0[None, :],
        Wblk, bblk, Wout, bout[None, :],
    )
    scale = 0.5 / (_DSM * B * P * K)
    return (out[0, 0] * scale).astype(jnp.float32)


# speedup vs baseline: 1.4075x; 1.2333x over previous
import functools

import jax
import jax.numpy as jnp
from jax.experimental import pallas as pl
from jax.experimental.pallas import tpu as pltpu

_DSM = 0.01
_BIG = 3.0e38
_LANES = 128


def _top4_values(d2, n_slices):
    R = d2.shape[0]
    m1 = jnp.full((R, _LANES), _BIG, jnp.float32)
    m2, m3, m4 = m1, m1, m1
    for s in range(n_slices):
        x = d2[:, s * _LANES:(s + 1) * _LANES]
        o1 = jnp.maximum(m1, x)
        m1 = jnp.minimum(m1, x)
        o2 = jnp.maximum(m2, o1)
        m2 = jnp.minimum(m2, o1)
        o3 = jnp.maximum(m3, o2)
        m3 = jnp.minimum(m3, o2)
        m4 = jnp.minimum(m4, o3)
    cat = jnp.concatenate([m1, m2, m3, m4], axis=1)
    iota_cat = jax.lax.broadcasted_iota(jnp.int32, cat.shape, 1)
    vs = []
    for j in range(4):
        mj = jnp.min(cat, axis=1, keepdims=True)
        vs.append(mj)
        if j < 3:
            isel = jnp.min(jnp.where(cat <= mj, iota_cat, jnp.int32(2**30)),
                           axis=1, keepdims=True)
            cat = jnp.where(iota_cat == isel, _BIG, cat)
    return vs


def _lanscore_kernel(
    pts_n_ref, ptsT_n_ref, pts_c_ref, ptsT_c_ref, pnt_oh_ref,
    W1_ref, b1_ref, W2_ref, b2_ref, Wx_ref, Wc_ref, b0_ref,
    Wblk_ref, bblk_ref, Wout_ref, bout_ref,
    out_ref,
    frames_scr, noise_scr,
    *, n_valid, P, K, C, Q_CHUNK,
):
    pts_n = pts_n_ref[0]
    ptsT_n = ptsT_n_ref[0]
    pts_c = pts_c_ref[0]
    ptsT_c = ptsT_c_ref[0]
    npad = pts_n.shape[0]
    m_clean = pts_c.shape[0]

    pnt_oh = pnt_oh_ref[...]
    q = jnp.dot(pnt_oh, pts_n, preferred_element_type=jnp.float32)
    h1 = jnp.maximum(
        jnp.dot(q, W1_ref[...], preferred_element_type=jnp.float32)
        + b1_ref[...], 0.0)
    feat = (jnp.dot(h1, W2_ref[...], preferred_element_type=jnp.float32)
            + b2_ref[...])

    pp_n = jnp.sum(ptsT_n * ptsT_n, axis=0, keepdims=True)
    col1_n = jax.lax.broadcasted_iota(jnp.int32, (1, npad), 1)
    if n_valid < npad:
        pp_n = jnp.where(col1_n >= n_valid, _BIG, pp_n)
    ptsT_na = jnp.concatenate([ptsT_n, pp_n], axis=0)
    qa = jnp.concatenate([q * -2.0, jnp.ones((P, 1), jnp.float32)], axis=1)
    d2n = jnp.dot(qa, ptsT_na, preferred_element_type=jnp.float32)

    n_slices_n = npad // _LANES
    for rnd in range(K // 4):
        vs = _top4_values(d2n, n_slices_n)
        ohf4 = jnp.concatenate(
            [(d2n == v).astype(jnp.float32) for v in vs], axis=0)
        frames_scr[rnd * 4 * P:(rnd + 1) * 4 * P, :] = jnp.dot(
            ohf4, pts_n, preferred_element_type=jnp.float32)
        d2n = jnp.where(d2n <= vs[3], _BIG, d2n)
    frames = frames_scr[...]

    pp_c = jnp.sum(ptsT_c * ptsT_c, axis=0, keepdims=True)
    ptsT_ca = jnp.concatenate([ptsT_c, pp_c], axis=0)
    pts_c1 = jnp.concatenate(
        [pts_c, jnp.ones((m_clean, 1), jnp.float32)], axis=1)
    n_rows = K * P
    n_slices = m_clean // _LANES

    def _chunk_body(ci, carry):
        fc = frames_scr[pl.ds(ci * Q_CHUNK, Q_CHUNK), :]
        fca = jnp.concatenate(
            [fc * -2.0, jnp.ones((Q_CHUNK, 1), jnp.float32)], axis=1)
        d2c = jnp.dot(fca, ptsT_ca,
                      preferred_element_type=jnp.float32)
        v4 = _top4_values(d2c, n_slices)[3]
        ohf = (d2c <= v4).astype(jnp.float32)
        acc = jnp.dot(ohf, pts_c1, preferred_element_type=jnp.float32)
        noise_scr[pl.ds(ci * Q_CHUNK, Q_CHUNK), :] = (
            fc - acc[:, :3] / acc[:, 3:4])
        return carry

    jax.lax.fori_loop(0, n_rows // Q_CHUNK, _chunk_body, 0)
    noise = noise_scr[...]

    q_t = jnp.concatenate([q] * K, axis=0)
    x = frames - q_t
    cwc = jnp.dot(feat, Wc_ref[...], preferred_element_type=jnp.float32)
    cwc_t = jnp.concatenate([cwc] * K, axis=0)
    h = jnp.maximum(
        jnp.dot(x, Wx_ref[...], preferred_element_type=jnp.float32)
        + cwc_t + b0_ref[...], 0.0)
    for i in range(Wblk_ref.shape[0]):
        h = jnp.maximum(
            jnp.dot(h, Wblk_ref[i], preferred_element_type=jnp.float32)
            + bblk_ref[i][None, :], 0.0) + h
    gp = (jnp.dot(h, Wout_ref[...], preferred_element_type=jnp.float32)
          + bout_ref[...])

    diff = -noise - gp
    s = jnp.sum(diff * diff).reshape(1, 1)
    @pl.when(pl.program_id(0) == 0)
    def _():
        out_ref[...] = jnp.zeros_like(out_ref)
    out_ref[...] += s


def kernel(pcl_noisy, pcl_clean, W1, b1, W2, b2, Wx, Wc, b0, Wblk, bblk,
           Wout, bout, pnt_idx):
    B, N, _ = pcl_noisy.shape
    M = pcl_clean.shape[1]
    P = pnt_idx.shape[0]
    K = 32
    C = 4
    Q_CHUNK = 128

    npad = ((N + _LANES - 1) // _LANES) * _LANES
    pts_n = jnp.pad(pcl_noisy, ((0, 0), (0, npad - N), (0, 0)))
    ptsT_n = jnp.transpose(pts_n, (0, 2, 1))
    pts_c = pcl_clean
    ptsT_c = jnp.transpose(pcl_clean, (0, 2, 1))
    pnt_oh = (pnt_idx[:, None] ==
              jnp.arange(npad, dtype=pnt_idx.dtype)[None, :]
              ).astype(jnp.float32)

    bspec = lambda shp, imap: pl.BlockSpec(shp, imap)
    per_b = lambda *shp: pl.BlockSpec((1,) + shp, lambda b: (b,) + (0,) * len(shp))
    fixed = lambda *shp: pl.BlockSpec(shp, lambda b: (0,) * len(shp))

    out = pl.pallas_call(
        functools.partial(_lanscore_kernel, n_valid=N, P=P, K=K, C=C,
                          Q_CHUNK=Q_CHUNK),
        grid=(B,),
        in_specs=[
            per_b(npad, 3), per_b(3, npad), per_b(M, 3), per_b(3, M),
            fixed(P, npad),
            fixed(3, W1.shape[1]), fixed(1, b1.shape[0]),
            fixed(*W2.shape), fixed(1, b2.shape[0]),
            fixed(*Wx.shape), fixed(*Wc.shape), fixed(1, b0.shape[0]),
            fixed(*Wblk.shape), fixed(*bblk.shape),
            fixed(*Wout.shape), fixed(1, bout.shape[0]),
        ],
        out_specs=pl.BlockSpec((1, 1), lambda b: (0, 0)),
        out_shape=jax.ShapeDtypeStruct((1, 1), jnp.float32),
        scratch_shapes=[
            pltpu.VMEM((K * P, 3), jnp.float32),
            pltpu.VMEM((K * P, 3), jnp.float32),
        ],
        compiler_params=pltpu.CompilerParams(
            dimension_semantics=("arbitrary",),
        ),
    )(
        pts_n, ptsT_n, pts_c, ptsT_c, pnt_oh,
        W1, b1[None, :], W2, b2[None, :], Wx, Wc, b0[None, :],
        Wblk, bblk, Wout, bout[None, :],
    )
    scale = 0.5 / (_DSM * B * P * K)
    return (out[0, 0] * scale).astype(jnp.float32)
